# baseline (speedup 1.0000x reference)
import jax
import jax.numpy as jnp
from jax import lax
from jax.experimental import pallas as pl
from jax.experimental.pallas import tpu as pltpu
from jax.experimental.pallas import tpu_sc as plsc

NC = 2
NS = 16
NW = NC * NS

B = 4096
L = 50
EMBED = 128
NSPLIT = 2
BK = B // NSPLIT
BPW = BK // NW
GRP = 4
NGRP = BPW // GRP
NBUF = 2


def _emb_body(idx_hbm, tab_hbm, out_hbm, idx_v, buf0, buf1, gsem, ssem):
    wid = lax.axis_index("s") * NC + lax.axis_index("c")
    bufs = (buf0, buf1)
    bat_base = wid * BPW

    pltpu.sync_copy(idx_hbm.at[wid], idx_v)

    def gstart(g, b):
        for k in range(GRP):
            pltpu.async_copy(
                tab_hbm.at[idx_v.at[g * GRP + k]], bufs[b].at[k], gsem)

    def body(g, b):
        for k in range(GRP):
            pltpu.make_async_copy(
                tab_hbm.at[idx_v.at[g * GRP + k]], bufs[b].at[k],
                gsem).wait()
        dst = out_hbm.at[pl.ds(bat_base + g * GRP, GRP)]
        pltpu.async_copy(bufs[b], dst, ssem)
        pltpu.make_async_copy(bufs[b], dst, ssem).wait()

    for b in range(NBUF):
        gstart(b, b)

    @pl.loop(0, NGRP - NBUF, step=NBUF)
    def _(g0):
        for b in range(NBUF):
            body(g0 + b, b)
            gstart(g0 + b + NBUF, b)

    for b in range(NBUF):
        body(NGRP - NBUF + b, b)


def _make_lookup():
    mesh = plsc.VectorSubcoreMesh(
        core_axis_name="c", subcore_axis_name="s",
        num_cores=NC, num_subcores=NS)
    return pl.kernel(
        _emb_body,
        out_type=jax.ShapeDtypeStruct((BK, L, EMBED), jnp.float32),
        mesh=mesh,
        scratch_types=[pltpu.VMEM((BPW, L), jnp.int32)]
        + [pltpu.VMEM((GRP, L, EMBED), jnp.float32) for _ in range(NBUF)]
        + [pltpu.SemaphoreType.DMA, pltpu.SemaphoreType.DMA],
    )


@jax.jit
def _emb(src_idx, tgt_idx, src_tab, tgt_tab):
    lookup = _make_lookup()
    halves = []
    for idx, tab in ((src_idx, src_tab), (tgt_idx, tgt_tab)):
        for h in range(NSPLIT):
            part = lax.slice_in_dim(idx, h * BK, (h + 1) * BK, axis=0)
            halves.append(lookup(part.reshape(NW, BPW, L), tab))
    src_out = lax.concatenate(halves[:NSPLIT], 0)
    tgt_out = lax.concatenate(halves[NSPLIT:], 0)
    return src_out, tgt_out


def kernel(src_indices, tgt_indices, src_table, tgt_table):
    src_out, tgt_out = _emb(src_indices, tgt_indices, src_table, tgt_table)
    return (src_out, tgt_out)


# speedup vs baseline: 1.5759x; 1.5759x over previous
import jax
import jax.numpy as jnp
from jax import lax
from jax.experimental import pallas as pl
from jax.experimental.pallas import tpu as pltpu
from jax.experimental.pallas import tpu_sc as plsc

NC = 2
NS = 16
NW = NC * NS

B = 4096
L = 50
EMBED = 128
BK = B
BPW = BK // NW
GRP = 4
NGRP = BPW // GRP
NBUF = 2


def _emb_body(idx_hbm, tab_hbm, out_hbm, idx_v, buf0, buf1, gsem, ssem):
    wid = lax.axis_index("s") * NC + lax.axis_index("c")
    bufs = (buf0, buf1)
    bat_base = wid * BPW

    pltpu.sync_copy(idx_hbm.at[wid], idx_v)

    def gstart(g, b):
        for k in range(GRP):
            pltpu.async_copy(
                tab_hbm.at[idx_v.at[g * GRP + k]], bufs[b].at[k], gsem)

    def body(g, b):
        for k in range(GRP):
            pltpu.make_async_copy(
                tab_hbm.at[idx_v.at[g * GRP + k]], bufs[b].at[k],
                gsem).wait()
        dst = out_hbm.at[pl.ds(bat_base + g * GRP, GRP)]
        pltpu.async_copy(bufs[b], dst, ssem)
        pltpu.make_async_copy(bufs[b], dst, ssem).wait()

    for b in range(NBUF):
        gstart(b, b)

    @pl.loop(0, NGRP - NBUF, step=NBUF)
    def _(g0):
        for b in range(NBUF):
            body(g0 + b, b)
            gstart(g0 + b + NBUF, b)

    for b in range(NBUF):
        body(NGRP - NBUF + b, b)


def _make_lookup():
    mesh = plsc.VectorSubcoreMesh(
        core_axis_name="c", subcore_axis_name="s",
        num_cores=NC, num_subcores=NS)
    return pl.kernel(
        _emb_body,
        out_type=jax.ShapeDtypeStruct((BK, L, EMBED), jnp.float32),
        mesh=mesh,
        scratch_types=[pltpu.VMEM((BPW, L), jnp.int32)]
        + [pltpu.VMEM((GRP, L, EMBED), jnp.float32) for _ in range(NBUF)]
        + [pltpu.SemaphoreType.DMA, pltpu.SemaphoreType.DMA],
    )


@jax.jit
def _emb(src_idx, tgt_idx, src_tab, tgt_tab):
    lookup = _make_lookup()
    src_out = lookup(src_idx.reshape(NW, BPW, L), src_tab)
    tgt_out = lookup(tgt_idx.reshape(NW, BPW, L), tgt_tab)
    return src_out, tgt_out


def kernel(src_indices, tgt_indices, src_table, tgt_table):
    src_out, tgt_out = _emb(src_indices, tgt_indices, src_table, tgt_table)
    return (src_out, tgt_out)
